# baseline (speedup 1.0000x reference)
import functools

import jax
import jax.numpy as jnp
from jax import lax
from jax.experimental import pallas as pl
from jax.experimental.pallas import tpu as pltpu
from jax.experimental.pallas import tpu_sc as plsc

VOCAB = 100000
EMBED = 200
BATCH = 1024
CTX = 50

NC = 2
NS = 16
NW = NC * NS

B_PER_W = BATCH // NW
G = 100
NCHUNK = (B_PER_W * CTX) // G
B_PER_SC = BATCH // NC


def _sc_gather_sum(x_r, tgt_r, zeros, emb_table):
    mesh = plsc.VectorSubcoreMesh(core_axis_name="c", subcore_axis_name="s",
                                  num_cores=NC, num_subcores=NS)

    @functools.partial(
        pl.kernel,
        out_type=jax.ShapeDtypeStruct((BATCH, EMBED), jnp.float32),
        mesh=mesh,
        compiler_params=pltpu.CompilerParams(use_tc_tiling_on_sc=False),
        scratch_types=[
            pltpu.VMEM((NCHUNK, G), jnp.int32),
            pltpu.VMEM((NCHUNK, G), jnp.int32),
            pltpu.VMEM((G, EMBED), jnp.float32),
            pltpu.VMEM((G, EMBED), jnp.float32),
            pltpu.VMEM_SHARED((B_PER_SC, EMBED), jnp.float32),
            pltpu.SemaphoreType.DMA,
            pltpu.SemaphoreType.DMA,
        ],
    )
    def body(x_hbm, tgt_hbm, zeros_hbm, table_hbm, s_hbm,
             idx_v, tgt_v, buf0, buf1, acc, sem0, sem1):
        c = lax.axis_index("c")
        s = lax.axis_index("s")
        pltpu.sync_copy(zeros_hbm, acc.at[pl.ds(s * B_PER_W, B_PER_W)])
        pltpu.sync_copy(x_hbm.at[c, s], idx_v)
        pltpu.sync_copy(tgt_hbm.at[s], tgt_v)

        bufs = (buf0, buf1)
        sems = (sem0, sem1)
        copies = [None, None]
        copies[0] = pltpu.async_copy(table_hbm.at[idx_v.at[0]], bufs[0], sems[0])
        for j in range(NCHUNK):
            if j + 1 < NCHUNK:
                copies[(j + 1) % 2] = pltpu.async_copy(
                    table_hbm.at[idx_v.at[j + 1]], bufs[(j + 1) % 2],
                    sems[(j + 1) % 2])
            copies[j % 2].wait()
            pltpu.sync_copy(bufs[j % 2], acc.at[tgt_v.at[j]], add=True)
        pltpu.sync_copy(acc.at[pl.ds(s * B_PER_W, B_PER_W)],
                        s_hbm.at[pl.ds(c * B_PER_SC + s * B_PER_W, B_PER_W)])

    return body(x_r, tgt_r, zeros, emb_table)


TV = 2048


def _mm_body(s_ref, w_ref, b_ref, o_ref):
    o_ref[...] = lax.dot_general(
        s_ref[...], w_ref[...], (((1,), (1,)), ((), ())),
        preferred_element_type=jnp.float32) + b_ref[...]


def _projection(s, W, b2d):
    return pl.pallas_call(
        _mm_body,
        grid=(pl.cdiv(VOCAB, TV),),
        in_specs=[
            pl.BlockSpec((BATCH, EMBED), lambda i: (0, 0)),
            pl.BlockSpec((TV, EMBED), lambda i: (i, 0)),
            pl.BlockSpec((1, TV), lambda i: (0, i)),
        ],
        out_specs=pl.BlockSpec((BATCH, TV), lambda i: (0, i)),
        out_shape=jax.ShapeDtypeStruct((BATCH, VOCAB), jnp.float32),
    )(s, W, b2d)


def kernel(x, emb_table, W, b):
    table2 = jnp.pad(emb_table, ((0, 8), (0, 0)))
    x_r = x.astype(jnp.int32).reshape(NC, NS, NCHUNK, G)
    tgt_r = (jnp.arange(NS * NCHUNK * G, dtype=jnp.int32) // CTX).reshape(
        NS, NCHUNK, G)
    zeros = jnp.zeros((B_PER_W, EMBED), jnp.float32)
    s = _sc_gather_sum(x_r, tgt_r, zeros, table2)
    return _projection(s, W, b.reshape(1, VOCAB))


# speedup vs baseline: 1.0204x; 1.0204x over previous
import functools

import jax
import jax.numpy as jnp
from jax import lax
from jax.experimental import pallas as pl
from jax.experimental.pallas import tpu as pltpu
from jax.experimental.pallas import tpu_sc as plsc

VOCAB = 100000
EMBED = 200
DPAD = 256
BATCH = 1024
CTX = 50

NC = 2
NS = 16
NW = NC * NS

B_PER_W = BATCH // NW
G = 80
NCHUNK = (B_PER_W * CTX) // G
ROWS_TOTAL = BATCH * CTX


def _sc_gather(x_r, table_p):
    mesh = plsc.VectorSubcoreMesh(core_axis_name="c", subcore_axis_name="s",
                                  num_cores=NC, num_subcores=NS)

    @functools.partial(
        pl.kernel,
        out_type=jax.ShapeDtypeStruct((ROWS_TOTAL, DPAD), jnp.float32),
        mesh=mesh,
        compiler_params=pltpu.CompilerParams(use_tc_tiling_on_sc=True),
        scratch_types=[
            pltpu.VMEM((NCHUNK, G), jnp.int32),
            pltpu.VMEM((G, DPAD), jnp.float32),
            pltpu.VMEM((G, DPAD), jnp.float32),
            pltpu.SemaphoreType.DMA,
            pltpu.SemaphoreType.DMA,
        ],
    )
    def body(x_hbm, table_hbm, e_hbm, idx_v, buf0, buf1, sem0, sem1):
        c = lax.axis_index("c")
        s = lax.axis_index("s")
        pltpu.sync_copy(x_hbm.at[c, s], idx_v)
        base = (c * NS + s) * (B_PER_W * CTX)

        bufs = (buf0, buf1)
        sems = (sem0, sem1)
        copies = [None, None]
        copies[0] = pltpu.async_copy(table_hbm.at[idx_v.at[0]], bufs[0], sems[0])
        for j in range(NCHUNK):
            if j + 1 < NCHUNK:
                copies[(j + 1) % 2] = pltpu.async_copy(
                    table_hbm.at[idx_v.at[j + 1]], bufs[(j + 1) % 2],
                    sems[(j + 1) % 2])
            copies[j % 2].wait()
            pltpu.sync_copy(bufs[j % 2], e_hbm.at[pl.ds(base + j * G, G)])

    return body(x_r, table_p)


TV = 2048


def _mm_body(s_ref, w_ref, b_ref, o_ref):
    o_ref[...] = lax.dot_general(
        s_ref[...], w_ref[...], (((1,), (1,)), ((), ())),
        preferred_element_type=jnp.float32) + b_ref[...]


def _projection(s, W, b2d):
    return pl.pallas_call(
        _mm_body,
        grid=(pl.cdiv(VOCAB, TV),),
        in_specs=[
            pl.BlockSpec((BATCH, EMBED), lambda i: (0, 0)),
            pl.BlockSpec((TV, EMBED), lambda i: (i, 0)),
            pl.BlockSpec((1, TV), lambda i: (0, i)),
        ],
        out_specs=pl.BlockSpec((BATCH, TV), lambda i: (0, i)),
        out_shape=jax.ShapeDtypeStruct((BATCH, VOCAB), jnp.float32),
    )(s, W, b2d)


def kernel(x, emb_table, W, b):
    table_p = jnp.pad(emb_table, ((0, 0), (0, DPAD - EMBED)))
    x_r = x.astype(jnp.int32).reshape(NC, NS, NCHUNK, G)
    e = _sc_gather(x_r, table_p)
    s = e.reshape(BATCH, CTX, DPAD).sum(axis=1)[:, :EMBED]
    return _projection(s, W, b.reshape(1, VOCAB))


# speedup vs baseline: 1.5380x; 1.5073x over previous
import functools

import jax
import jax.numpy as jnp
from jax import lax
from jax.experimental import pallas as pl
from jax.experimental.pallas import tpu as pltpu
from jax.experimental.pallas import tpu_sc as plsc

VOCAB = 100000
EMBED = 200
DPAD = 256
BATCH = 1024
CTX = 50

NC = 2
NS = 16
NW = NC * NS

B_PER_W = BATCH // NW
G = 80
NCHUNK = (B_PER_W * CTX) // G
ROWS_TOTAL = BATCH * CTX

PR = 4000


def _pad_body(t_ref, o_ref):
    o_ref[...] = jnp.concatenate(
        [t_ref[...], jnp.zeros((PR, DPAD - EMBED), jnp.float32)], axis=1)


def _pad_table(emb_table):
    return pl.pallas_call(
        _pad_body,
        grid=(VOCAB // PR,),
        in_specs=[pl.BlockSpec((PR, EMBED), lambda i: (i, 0))],
        out_specs=pl.BlockSpec((PR, DPAD), lambda i: (i, 0)),
        out_shape=jax.ShapeDtypeStruct((VOCAB, DPAD), jnp.float32),
    )(emb_table)


def _sc_gather(x_r, table_p):
    mesh = plsc.VectorSubcoreMesh(core_axis_name="c", subcore_axis_name="s",
                                  num_cores=NC, num_subcores=NS)

    @functools.partial(
        pl.kernel,
        out_type=jax.ShapeDtypeStruct((ROWS_TOTAL, DPAD), jnp.float32),
        mesh=mesh,
        compiler_params=pltpu.CompilerParams(use_tc_tiling_on_sc=True),
        scratch_types=[
            pltpu.VMEM((NCHUNK, G), jnp.int32),
            pltpu.VMEM((G, DPAD), jnp.float32),
            pltpu.VMEM((G, DPAD), jnp.float32),
            pltpu.SemaphoreType.DMA,
            pltpu.SemaphoreType.DMA,
        ],
    )
    def body(x_hbm, table_hbm, e_hbm, idx_v, buf0, buf1, sem0, sem1):
        c = lax.axis_index("c")
        s = lax.axis_index("s")
        pltpu.sync_copy(x_hbm.at[c, s], idx_v)
        base = (c * NS + s) * (B_PER_W * CTX)

        bufs = (buf0, buf1)
        sems = (sem0, sem1)
        copies = [None, None]
        copies[0] = pltpu.async_copy(table_hbm.at[idx_v.at[0]], bufs[0], sems[0])
        for j in range(NCHUNK):
            if j + 1 < NCHUNK:
                copies[(j + 1) % 2] = pltpu.async_copy(
                    table_hbm.at[idx_v.at[j + 1]], bufs[(j + 1) % 2],
                    sems[(j + 1) % 2])
            copies[j % 2].wait()
            pltpu.sync_copy(bufs[j % 2], e_hbm.at[pl.ds(base + j * G, G)])

    return body(x_r, table_p)


BBLK = 64


def _sum_body(e_ref, s_ref):
    for i in range(BBLK):
        s_ref[i, :] = jnp.sum(e_ref[pl.ds(i * CTX, CTX), :], axis=0)


def _ctx_sum(e):
    return pl.pallas_call(
        _sum_body,
        grid=(BATCH // BBLK,),
        in_specs=[pl.BlockSpec((BBLK * CTX, DPAD), lambda i: (i, 0))],
        out_specs=pl.BlockSpec((BBLK, DPAD), lambda i: (i, 0)),
        out_shape=jax.ShapeDtypeStruct((BATCH, DPAD), jnp.float32),
    )(e)


TV = 2048


def _mm_body(s_ref, w_ref, b_ref, o_ref):
    o_ref[...] = lax.dot_general(
        s_ref[...], w_ref[...], (((1,), (1,)), ((), ())),
        preferred_element_type=jnp.float32) + b_ref[...]


def _projection(s, W, b2d):
    return pl.pallas_call(
        _mm_body,
        grid=(pl.cdiv(VOCAB, TV),),
        in_specs=[
            pl.BlockSpec((BATCH, EMBED), lambda i: (0, 0)),
            pl.BlockSpec((TV, EMBED), lambda i: (i, 0)),
            pl.BlockSpec((1, TV), lambda i: (0, i)),
        ],
        out_specs=pl.BlockSpec((BATCH, TV), lambda i: (0, i)),
        out_shape=jax.ShapeDtypeStruct((BATCH, VOCAB), jnp.float32),
    )(s, W, b2d)


def kernel(x, emb_table, W, b):
    table_p = _pad_table(emb_table)
    x_r = x.astype(jnp.int32).reshape(NC, NS, NCHUNK, G)
    e = _sc_gather(x_r, table_p)
    s = _ctx_sum(e)
    return _projection(s[:, :EMBED], W, b.reshape(1, VOCAB))
